# baseline (speedup 1.0000x reference)
import functools

import jax
import jax.numpy as jnp
from jax import lax
from jax.experimental import pallas as pl
from jax.experimental.pallas import tpu as pltpu
from jax.experimental.pallas import tpu_sc as plsc


_NC = 2
_NS = 16
_NW = _NC * _NS
_CHUNK = 16
_LANES = 16


def _sc_gather(emb, idx3d, n_idx, e_dim, out_w):
    view = emb.reshape(-1, 128)
    n_view = view.shape[0]
    rows_per_w = n_idx // _NW
    chunks_per_w = rows_per_w // _CHUNK
    vr_per_row = (127 + e_dim + 127) // 128
    nstage = vr_per_row * _CHUNK
    stage_words = nstage * 128
    mesh = plsc.VectorSubcoreMesh(core_axis_name="c", subcore_axis_name="s")

    @functools.partial(
        pl.kernel,
        mesh=mesh,
        out_type=jax.ShapeDtypeStruct((n_idx, out_w), jnp.float32),
        scratch_types=[
            pltpu.VMEM((chunks_per_w, _CHUNK), jnp.int32),
            pltpu.VMEM((nstage,), jnp.int32),
            pltpu.VMEM((nstage, 128), jnp.float32),
            pltpu.VMEM((_CHUNK, out_w), jnp.float32),
            pltpu.SemaphoreType.DMA,
        ],
        compiler_params=pltpu.CompilerParams(needs_layout_passes=False),
    )
    def gather_kernel(view_hbm, idx_hbm, out_hbm, idx_v, gbuf, staged, xbuf,
                      sem):
        wid = lax.axis_index("s") * _NC + lax.axis_index("c")
        base = wid * rows_per_w
        pltpu.sync_copy(idx_hbm.at[wid], idx_v)
        iota = lax.iota(jnp.int32, _LANES)
        for c in range(chunks_per_w):
            iv = idx_v[c]
            v0 = iv * e_dim
            r0 = lax.shift_right_logical(v0, 7)
            s0 = lax.bitwise_and(v0, 127)
            for part in range(vr_per_row):
                plsc.store_scatter(
                    gbuf, [iota * vr_per_row + part],
                    jnp.minimum(r0 + part, n_view - 1))
            pltpu.async_copy(view_hbm.at[gbuf], staged, sem).wait()

            g0 = iota * (vr_per_row * 128) + s0

            def body(j, g):
                gc = jnp.minimum(g, stage_words - 1)
                val = plsc.load_gather(
                    staged, [lax.shift_right_logical(gc, 7),
                             lax.bitwise_and(gc, 127)])
                plsc.store_scatter(xbuf, [iota, jnp.full((_LANES,), j,
                                                         jnp.int32)], val)
                return g + 1

            lax.fori_loop(0, out_w, body, g0)
            pltpu.sync_copy(xbuf, out_hbm.at[pl.ds(base + c * _CHUNK, _CHUNK)])

    return gather_kernel(view, idx3d)


def _input_matmul_body(x_ref, w_ref, b_ref, o_ref):
    o_ref[...] = (
        jnp.dot(x_ref[...], w_ref[...], preferred_element_type=jnp.float32)
        + b_ref[...]
    )


def _input_matmul(x, wcat, bcat, bm=256):
    m, k = x.shape
    n = wcat.shape[1]
    return pl.pallas_call(
        _input_matmul_body,
        grid=(m // bm,),
        in_specs=[
            pl.BlockSpec((bm, k), lambda i: (i, 0)),
            pl.BlockSpec((k, n), lambda i: (0, 0)),
            pl.BlockSpec((1, n), lambda i: (0, 0)),
        ],
        out_specs=pl.BlockSpec((bm, n), lambda i: (i, 0)),
        out_shape=jax.ShapeDtypeStruct((m, n), jnp.float32),
        compiler_params=pltpu.CompilerParams(
            dimension_semantics=("arbitrary",),
        ),
    )(x, wcat, bcat)


def _gru_scan_body(gi_f_ref, gi_b_ref, whtf_ref, whtb_ref, bnf_ref, bnb_ref,
                   fcw_ref, fcb_ref, out_ref, hf_ref, hb_ref):
    t = pl.program_id(0)
    nsteps = pl.num_programs(0)

    @pl.when(t == 0)
    def _init():
        hf_ref[...] = jnp.zeros_like(hf_ref)
        hb_ref[...] = jnp.zeros_like(hb_ref)

    def step(gi_ref, wht_ref, bn_ref, h_ref):
        h = h_ref[...]
        hdim = h.shape[1]
        gi = gi_ref[0]
        gh = jnp.dot(h, wht_ref[...], preferred_element_type=jnp.float32)
        r = jax.nn.sigmoid(gi[:, :hdim] + gh[:, :hdim])
        z = jax.nn.sigmoid(gi[:, hdim:2 * hdim] + gh[:, hdim:2 * hdim])
        n = jnp.tanh(gi[:, 2 * hdim:] + r * (gh[:, 2 * hdim:] + bn_ref[...]))
        h_new = (1.0 - z) * n + z * h
        h_ref[...] = h_new
        return h_new

    hf = step(gi_f_ref, whtf_ref, bnf_ref, hf_ref)
    hb = step(gi_b_ref, whtb_ref, bnb_ref, hb_ref)

    @pl.when(t == nsteps - 1)
    def _head():
        hdim = hf.shape[1]
        wf = fcw_ref[0, :hdim][None, :]
        wb = fcw_ref[0, hdim:][None, :]
        logit = (jnp.sum(hf * wf, axis=1, keepdims=True)
                 + jnp.sum(hb * wb, axis=1, keepdims=True)
                 + fcb_ref[0, 0])
        out_ref[...] = jax.nn.sigmoid(logit)


def _gru_scan(gi, whtf, whtb, bnf, bnb, fcw, fcb, t_len, b_dim, h_dim):
    g3 = 3 * h_dim
    return pl.pallas_call(
        _gru_scan_body,
        grid=(t_len,),
        in_specs=[
            pl.BlockSpec((1, b_dim, g3), lambda t: (t, 0, 0)),
            pl.BlockSpec((1, b_dim, g3), lambda t: (t_len - 1 - t, 0, 1)),
            pl.BlockSpec((h_dim, g3), lambda t: (0, 0)),
            pl.BlockSpec((h_dim, g3), lambda t: (0, 0)),
            pl.BlockSpec((1, h_dim), lambda t: (0, 0)),
            pl.BlockSpec((1, h_dim), lambda t: (0, 0)),
            pl.BlockSpec((1, 2 * h_dim), lambda t: (0, 0)),
            pl.BlockSpec((1, 1), lambda t: (0, 0)),
        ],
        out_specs=pl.BlockSpec((b_dim, 1), lambda t: (0, 0)),
        out_shape=jax.ShapeDtypeStruct((b_dim, 1), jnp.float32),
        scratch_shapes=[
            pltpu.VMEM((b_dim, h_dim), jnp.float32),
            pltpu.VMEM((b_dim, h_dim), jnp.float32),
        ],
        compiler_params=pltpu.CompilerParams(
            dimension_semantics=("arbitrary",),
        ),
    )(gi, gi, whtf, whtb, bnf, bnb, fcw, fcb)


def kernel(input, emb, Wih_f, Whh_f, bih_f, bhh_f, Wih_b, Whh_b, bih_b, bhh_b,
           fcW, fcb):
    b_dim, t_len = input.shape
    v_dim, e_dim = emb.shape
    h_dim = Whh_f.shape[1]
    n_idx = b_dim * t_len

    e_pad = (e_dim + 7) // 8 * 8
    idx3d = input.T.reshape(_NW, n_idx // (_NW * _CHUNK), _CHUNK).astype(jnp.int32)
    x = _sc_gather(emb, idx3d, n_idx, e_dim, e_pad)

    def fold_bias(bih, bhh):
        return jnp.concatenate(
            [bih[: 2 * h_dim] + bhh[: 2 * h_dim], bih[2 * h_dim:]])

    wcat = jnp.concatenate([Wih_f.T, Wih_b.T], axis=1)
    wcat = jnp.pad(wcat, ((0, e_pad - e_dim), (0, 0)))
    bcat = jnp.concatenate([fold_bias(bih_f, bhh_f),
                            fold_bias(bih_b, bhh_b)])[None, :]
    gi = _input_matmul(x, wcat, bcat)
    gi = gi.reshape(t_len, b_dim, 6 * h_dim)

    label = _gru_scan(
        gi,
        Whh_f.T, Whh_b.T,
        bhh_f[2 * h_dim:][None, :], bhh_b[2 * h_dim:][None, :],
        fcW, fcb.reshape(1, 1),
        t_len, b_dim, h_dim,
    )
    return jnp.squeeze(label, axis=1)


# speedup vs baseline: 1.9360x; 1.9360x over previous
import functools

import jax
import jax.numpy as jnp
from jax import lax
from jax.experimental import pallas as pl
from jax.experimental.pallas import tpu as pltpu
from jax.experimental.pallas import tpu_sc as plsc


_NC = 2
_NS = 16
_NW = _NC * _NS
_CHUNK = 80


def _pad_table_body(x_ref, o_ref):
    bm = x_ref.shape[0]
    o_ref[...] = jnp.concatenate(
        [x_ref[...], jnp.zeros((bm, 84), jnp.float32)], axis=1)


def _pad_table(emb, e_pad, bm=1000):
    v_rows, e_dim = emb.shape
    return pl.pallas_call(
        _pad_table_body,
        grid=(v_rows // bm,),
        in_specs=[pl.BlockSpec((bm, e_dim), lambda i: (i, 0))],
        out_specs=pl.BlockSpec((bm, e_pad), lambda i: (i, 0)),
        out_shape=jax.ShapeDtypeStruct((v_rows, e_pad), jnp.float32),
        compiler_params=pltpu.CompilerParams(
            dimension_semantics=("arbitrary",),
        ),
    )(emb)


def _sc_gather(emb_p, idx3d, n_idx, e_pad):
    rows_per_w = n_idx // _NW
    chunks_per_w = rows_per_w // _CHUNK
    mesh = plsc.VectorSubcoreMesh(core_axis_name="c", subcore_axis_name="s")

    @functools.partial(
        pl.kernel,
        mesh=mesh,
        out_type=jax.ShapeDtypeStruct((n_idx, e_pad), jnp.float32),
        scratch_types=[
            pltpu.VMEM((chunks_per_w, _CHUNK), jnp.int32),
            pltpu.VMEM((_CHUNK, e_pad), jnp.float32),
            pltpu.SemaphoreType.DMA,
        ],
    )
    def gather_kernel(emb_hbm, idx_hbm, out_hbm, idx_v, rows_v, sem):
        wid = lax.axis_index("s") * _NC + lax.axis_index("c")
        base = wid * rows_per_w
        pltpu.sync_copy(idx_hbm.at[wid], idx_v)
        for c in range(chunks_per_w):
            pltpu.async_copy(emb_hbm.at[idx_v.at[c]], rows_v, sem).wait()
            pltpu.sync_copy(rows_v, out_hbm.at[pl.ds(base + c * _CHUNK, _CHUNK)])

    return gather_kernel(emb_p, idx3d)


def _input_matmul_body(x_ref, w_ref, b_ref, o_ref):
    o_ref[...] = (
        jnp.dot(x_ref[...], w_ref[...], preferred_element_type=jnp.float32)
        + b_ref[...]
    )


def _input_matmul(x, wcat, bcat, bm=256):
    m, k = x.shape
    n = wcat.shape[1]
    return pl.pallas_call(
        _input_matmul_body,
        grid=(m // bm,),
        in_specs=[
            pl.BlockSpec((bm, k), lambda i: (i, 0)),
            pl.BlockSpec((k, n), lambda i: (0, 0)),
            pl.BlockSpec((1, n), lambda i: (0, 0)),
        ],
        out_specs=pl.BlockSpec((bm, n), lambda i: (i, 0)),
        out_shape=jax.ShapeDtypeStruct((m, n), jnp.float32),
        compiler_params=pltpu.CompilerParams(
            dimension_semantics=("arbitrary",),
        ),
    )(x, wcat, bcat)


def _gru_scan_body(gi_f_ref, gi_b_ref, whtf_ref, whtb_ref, bnf_ref, bnb_ref,
                   fcw_ref, fcb_ref, out_ref, hf_ref, hb_ref):
    t = pl.program_id(0)
    nsteps = pl.num_programs(0)

    @pl.when(t == 0)
    def _init():
        hf_ref[...] = jnp.zeros_like(hf_ref)
        hb_ref[...] = jnp.zeros_like(hb_ref)

    def step(gi_ref, wht_ref, bn_ref, h_ref):
        h = h_ref[...]
        hdim = h.shape[1]
        gi = gi_ref[0]
        gh = jnp.dot(h, wht_ref[...], preferred_element_type=jnp.float32)
        r = jax.nn.sigmoid(gi[:, :hdim] + gh[:, :hdim])
        z = jax.nn.sigmoid(gi[:, hdim:2 * hdim] + gh[:, hdim:2 * hdim])
        n = jnp.tanh(gi[:, 2 * hdim:] + r * (gh[:, 2 * hdim:] + bn_ref[...]))
        h_new = (1.0 - z) * n + z * h
        h_ref[...] = h_new
        return h_new

    hf = step(gi_f_ref, whtf_ref, bnf_ref, hf_ref)
    hb = step(gi_b_ref, whtb_ref, bnb_ref, hb_ref)

    @pl.when(t == nsteps - 1)
    def _head():
        hdim = hf.shape[1]
        wf = fcw_ref[0, :hdim][None, :]
        wb = fcw_ref[0, hdim:][None, :]
        logit = (jnp.sum(hf * wf, axis=1, keepdims=True)
                 + jnp.sum(hb * wb, axis=1, keepdims=True)
                 + fcb_ref[0, 0])
        out_ref[...] = jax.nn.sigmoid(logit)


def _gru_scan(gi, whtf, whtb, bnf, bnb, fcw, fcb, t_len, b_dim, h_dim):
    g3 = 3 * h_dim
    return pl.pallas_call(
        _gru_scan_body,
        grid=(t_len,),
        in_specs=[
            pl.BlockSpec((1, b_dim, g3), lambda t: (t, 0, 0)),
            pl.BlockSpec((1, b_dim, g3), lambda t: (t_len - 1 - t, 0, 1)),
            pl.BlockSpec((h_dim, g3), lambda t: (0, 0)),
            pl.BlockSpec((h_dim, g3), lambda t: (0, 0)),
            pl.BlockSpec((1, h_dim), lambda t: (0, 0)),
            pl.BlockSpec((1, h_dim), lambda t: (0, 0)),
            pl.BlockSpec((1, 2 * h_dim), lambda t: (0, 0)),
            pl.BlockSpec((1, 1), lambda t: (0, 0)),
        ],
        out_specs=pl.BlockSpec((b_dim, 1), lambda t: (0, 0)),
        out_shape=jax.ShapeDtypeStruct((b_dim, 1), jnp.float32),
        scratch_shapes=[
            pltpu.VMEM((b_dim, h_dim), jnp.float32),
            pltpu.VMEM((b_dim, h_dim), jnp.float32),
        ],
        compiler_params=pltpu.CompilerParams(
            dimension_semantics=("arbitrary",),
        ),
    )(gi, gi, whtf, whtb, bnf, bnb, fcw, fcb)


def kernel(input, emb, Wih_f, Whh_f, bih_f, bhh_f, Wih_b, Whh_b, bih_b, bhh_b,
           fcW, fcb):
    b_dim, t_len = input.shape
    v_dim, e_dim = emb.shape
    h_dim = Whh_f.shape[1]
    n_idx = b_dim * t_len

    e_pad = (e_dim + 127) // 128 * 128
    emb_p = _pad_table(emb, e_pad)
    idx3d = input.T.reshape(_NW, n_idx // (_NW * _CHUNK), _CHUNK).astype(jnp.int32)
    x = _sc_gather(emb_p, idx3d, n_idx, e_pad)

    def fold_bias(bih, bhh):
        return jnp.concatenate(
            [bih[: 2 * h_dim] + bhh[: 2 * h_dim], bih[2 * h_dim:]])

    wcat = jnp.concatenate([Wih_f.T, Wih_b.T], axis=1)
    wcat = jnp.pad(wcat, ((0, e_pad - e_dim), (0, 0)))
    bcat = jnp.concatenate([fold_bias(bih_f, bhh_f),
                            fold_bias(bih_b, bhh_b)])[None, :]
    gi = _input_matmul(x, wcat, bcat)
    gi = gi.reshape(t_len, b_dim, 6 * h_dim)

    label = _gru_scan(
        gi,
        Whh_f.T, Whh_b.T,
        bhh_f[2 * h_dim:][None, :], bhh_b[2 * h_dim:][None, :],
        fcW, fcb.reshape(1, 1),
        t_len, b_dim, h_dim,
    )
    return jnp.squeeze(label, axis=1)


# speedup vs baseline: 2.0577x; 1.0629x over previous
import functools

import jax
import jax.numpy as jnp
from jax import lax
from jax.experimental import pallas as pl
from jax.experimental.pallas import tpu as pltpu
from jax.experimental.pallas import tpu_sc as plsc


_NC = 2
_NS = 16
_NW = _NC * _NS
_CHUNK = 80


def _pad_table_body(x_ref, o_ref):
    bm = x_ref.shape[0]
    o_ref[...] = jnp.concatenate(
        [x_ref[...], jnp.zeros((bm, 84), jnp.float32)], axis=1)


def _pad_table(emb, e_pad, bm=1000):
    v_rows, e_dim = emb.shape
    return pl.pallas_call(
        _pad_table_body,
        grid=(v_rows // bm,),
        in_specs=[pl.BlockSpec((bm, e_dim), lambda i: (i, 0))],
        out_specs=pl.BlockSpec((bm, e_pad), lambda i: (i, 0)),
        out_shape=jax.ShapeDtypeStruct((v_rows, e_pad), jnp.float32),
        compiler_params=pltpu.CompilerParams(
            dimension_semantics=("arbitrary",),
        ),
    )(emb)


def _sc_gather(emb_p, idx3d, n_idx, e_pad):
    rows_per_w = n_idx // _NW
    chunks_per_w = rows_per_w // _CHUNK
    mesh = plsc.VectorSubcoreMesh(core_axis_name="c", subcore_axis_name="s")

    @functools.partial(
        pl.kernel,
        mesh=mesh,
        out_type=jax.ShapeDtypeStruct((n_idx, e_pad), jnp.float32),
        scratch_types=[
            pltpu.VMEM((chunks_per_w, _CHUNK), jnp.int32),
            pltpu.VMEM((_CHUNK, e_pad), jnp.float32),
            pltpu.SemaphoreType.DMA,
        ],
    )
    def gather_kernel(emb_hbm, idx_hbm, out_hbm, idx_v, rows_v, sem):
        wid = lax.axis_index("s") * _NC + lax.axis_index("c")
        base = wid * rows_per_w
        pltpu.sync_copy(idx_hbm.at[wid], idx_v)
        for c in range(chunks_per_w):
            pltpu.async_copy(emb_hbm.at[idx_v.at[c]], rows_v, sem).wait()
            pltpu.sync_copy(rows_v, out_hbm.at[pl.ds(base + c * _CHUNK, _CHUNK)])

    return gather_kernel(emb_p, idx3d)


def _input_matmul_body(x_ref, w_ref, b_ref, o_ref):
    o_ref[...] = (
        jnp.dot(x_ref[...], w_ref[...], preferred_element_type=jnp.float32)
        + b_ref[...]
    ).astype(jnp.bfloat16)


def _input_matmul(x, wcat, bcat, bm=256):
    m, k = x.shape
    n = wcat.shape[1]
    return pl.pallas_call(
        _input_matmul_body,
        grid=(m // bm,),
        in_specs=[
            pl.BlockSpec((bm, k), lambda i: (i, 0)),
            pl.BlockSpec((k, n), lambda i: (0, 0)),
            pl.BlockSpec((1, n), lambda i: (0, 0)),
        ],
        out_specs=pl.BlockSpec((bm, n), lambda i: (i, 0)),
        out_shape=jax.ShapeDtypeStruct((m, n), jnp.bfloat16),
        compiler_params=pltpu.CompilerParams(
            dimension_semantics=("arbitrary",),
        ),
    )(x, wcat, bcat)


def _gru_scan_body(gi_f_ref, gi_b_ref, whtf_ref, whtb_ref, bnf_ref, bnb_ref,
                   fcw_ref, fcb_ref, out_ref, hf_ref, hb_ref):
    t = pl.program_id(0)
    nsteps = pl.num_programs(0)

    @pl.when(t == 0)
    def _init():
        hf_ref[...] = jnp.zeros_like(hf_ref)
        hb_ref[...] = jnp.zeros_like(hb_ref)

    def step(gi_ref, wht_ref, bn_ref, h_ref):
        h = h_ref[...]
        hdim = h.shape[1]
        gi = gi_ref[0].astype(jnp.float32)
        gh = jnp.dot(h.astype(jnp.bfloat16), wht_ref[...],
                     preferred_element_type=jnp.float32)
        r = jax.nn.sigmoid(gi[:, :hdim] + gh[:, :hdim])
        z = jax.nn.sigmoid(gi[:, hdim:2 * hdim] + gh[:, hdim:2 * hdim])
        n = jnp.tanh(gi[:, 2 * hdim:] + r * (gh[:, 2 * hdim:] + bn_ref[...]))
        h_new = (1.0 - z) * n + z * h
        h_ref[...] = h_new
        return h_new

    hf = step(gi_f_ref, whtf_ref, bnf_ref, hf_ref)
    hb = step(gi_b_ref, whtb_ref, bnb_ref, hb_ref)

    @pl.when(t == nsteps - 1)
    def _head():
        hdim = hf.shape[1]
        wf = fcw_ref[0, :hdim][None, :]
        wb = fcw_ref[0, hdim:][None, :]
        logit = (jnp.sum(hf * wf, axis=1, keepdims=True)
                 + jnp.sum(hb * wb, axis=1, keepdims=True)
                 + fcb_ref[0, 0])
        out_ref[...] = jax.nn.sigmoid(logit)


def _gru_scan(gi, whtf, whtb, bnf, bnb, fcw, fcb, t_len, b_dim, h_dim):
    g3 = 3 * h_dim
    return pl.pallas_call(
        _gru_scan_body,
        grid=(t_len,),
        in_specs=[
            pl.BlockSpec((1, b_dim, g3), lambda t: (t, 0, 0)),
            pl.BlockSpec((1, b_dim, g3), lambda t: (t_len - 1 - t, 0, 1)),
            pl.BlockSpec((h_dim, g3), lambda t: (0, 0)),
            pl.BlockSpec((h_dim, g3), lambda t: (0, 0)),
            pl.BlockSpec((1, h_dim), lambda t: (0, 0)),
            pl.BlockSpec((1, h_dim), lambda t: (0, 0)),
            pl.BlockSpec((1, 2 * h_dim), lambda t: (0, 0)),
            pl.BlockSpec((1, 1), lambda t: (0, 0)),
        ],
        out_specs=pl.BlockSpec((b_dim, 1), lambda t: (0, 0)),
        out_shape=jax.ShapeDtypeStruct((b_dim, 1), jnp.float32),
        scratch_shapes=[
            pltpu.VMEM((b_dim, h_dim), jnp.float32),
            pltpu.VMEM((b_dim, h_dim), jnp.float32),
        ],
        compiler_params=pltpu.CompilerParams(
            dimension_semantics=("arbitrary",),
        ),
    )(gi, gi, whtf, whtb, bnf, bnb, fcw, fcb)


def kernel(input, emb, Wih_f, Whh_f, bih_f, bhh_f, Wih_b, Whh_b, bih_b, bhh_b,
           fcW, fcb):
    b_dim, t_len = input.shape
    v_dim, e_dim = emb.shape
    h_dim = Whh_f.shape[1]
    n_idx = b_dim * t_len

    e_pad = (e_dim + 127) // 128 * 128
    emb_p = _pad_table(emb, e_pad)
    idx3d = input.T.reshape(_NW, n_idx // (_NW * _CHUNK), _CHUNK).astype(jnp.int32)
    x = _sc_gather(emb_p, idx3d, n_idx, e_pad)

    def fold_bias(bih, bhh):
        return jnp.concatenate(
            [bih[: 2 * h_dim] + bhh[: 2 * h_dim], bih[2 * h_dim:]])

    wcat = jnp.concatenate([Wih_f.T, Wih_b.T], axis=1)
    wcat = jnp.pad(wcat, ((0, e_pad - e_dim), (0, 0)))
    bcat = jnp.concatenate([fold_bias(bih_f, bhh_f),
                            fold_bias(bih_b, bhh_b)])[None, :]
    gi = _input_matmul(x, wcat, bcat)
    gi = gi.reshape(t_len, b_dim, 6 * h_dim)

    label = _gru_scan(
        gi,
        Whh_f.T.astype(jnp.bfloat16), Whh_b.T.astype(jnp.bfloat16),
        bhh_f[2 * h_dim:][None, :], bhh_b[2 * h_dim:][None, :],
        fcW, fcb.reshape(1, 1),
        t_len, b_dim, h_dim,
    )
    return jnp.squeeze(label, axis=1)


# speedup vs baseline: 2.2378x; 1.0875x over previous
import functools

import jax
import jax.numpy as jnp
from jax import lax
from jax.experimental import pallas as pl
from jax.experimental.pallas import tpu as pltpu
from jax.experimental.pallas import tpu_sc as plsc


_NC = 2
_NS = 16
_NW = _NC * _NS
_CHUNK = 80


def _pad_table_body(x_ref, o_ref):
    bm = x_ref.shape[0]
    o_ref[...] = jnp.concatenate(
        [x_ref[...], jnp.zeros((bm, 84), jnp.float32)], axis=1)


def _pad_table(emb, e_pad, bm=1000):
    v_rows, e_dim = emb.shape
    return pl.pallas_call(
        _pad_table_body,
        grid=(v_rows // bm,),
        in_specs=[pl.BlockSpec((bm, e_dim), lambda i: (i, 0))],
        out_specs=pl.BlockSpec((bm, e_pad), lambda i: (i, 0)),
        out_shape=jax.ShapeDtypeStruct((v_rows, e_pad), jnp.float32),
        compiler_params=pltpu.CompilerParams(
            dimension_semantics=("arbitrary",),
        ),
    )(emb)


def _sc_gather(emb_p, idx3d, n_idx, e_pad):
    rows_per_w = n_idx // _NW
    chunks_per_w = rows_per_w // _CHUNK
    mesh = plsc.VectorSubcoreMesh(core_axis_name="c", subcore_axis_name="s")

    @functools.partial(
        pl.kernel,
        mesh=mesh,
        out_type=jax.ShapeDtypeStruct((n_idx, e_pad), jnp.float32),
        scratch_types=[
            pltpu.VMEM((chunks_per_w, _CHUNK), jnp.int32),
            pltpu.VMEM((_CHUNK, e_pad), jnp.float32),
            pltpu.SemaphoreType.DMA,
        ],
    )
    def gather_kernel(emb_hbm, idx_hbm, out_hbm, idx_v, rows_v, sem):
        wid = lax.axis_index("s") * _NC + lax.axis_index("c")
        base = wid * rows_per_w
        pltpu.sync_copy(idx_hbm.at[wid], idx_v)
        for c in range(chunks_per_w):
            pltpu.async_copy(emb_hbm.at[idx_v.at[c]], rows_v, sem).wait()
            pltpu.sync_copy(rows_v, out_hbm.at[pl.ds(base + c * _CHUNK, _CHUNK)])

    return gather_kernel(emb_p, idx3d)


def _input_matmul_body(x_ref, w_ref, b_ref, o_ref):
    o_ref[...] = (
        jnp.dot(x_ref[...].astype(jnp.bfloat16), w_ref[...],
                preferred_element_type=jnp.float32)
        + b_ref[...]
    ).astype(jnp.bfloat16)


def _input_matmul(x, wcat, bcat, bm=256):
    m, k = x.shape
    kw, n = wcat.shape
    return pl.pallas_call(
        _input_matmul_body,
        grid=(m // bm,),
        in_specs=[
            pl.BlockSpec((bm, k), lambda i: (i, 0)),
            pl.BlockSpec((kw, n), lambda i: (0, 0)),
            pl.BlockSpec((1, n), lambda i: (0, 0)),
        ],
        out_specs=pl.BlockSpec((bm, n), lambda i: (i, 0)),
        out_shape=jax.ShapeDtypeStruct((m, n), jnp.bfloat16),
        compiler_params=pltpu.CompilerParams(
            dimension_semantics=("arbitrary",),
        ),
    )(x, wcat, bcat)


def _gru_scan_body(gi_f_ref, gi_b_ref, whtf_ref, whtb_ref, bnf_ref, bnb_ref,
                   fcw_ref, fcb_ref, out_ref, hf_ref, hb_ref):
    t = pl.program_id(0)
    nsteps = pl.num_programs(0)

    @pl.when(t == 0)
    def _init():
        hf_ref[...] = jnp.zeros_like(hf_ref)
        hb_ref[...] = jnp.zeros_like(hb_ref)

    def gates(gi, gh, bn_ref, h):
        hdim = h.shape[1]
        gi = gi.astype(jnp.float32)
        r = jax.nn.sigmoid(gi[:, :hdim] + gh[:, :hdim])
        z = jax.nn.sigmoid(gi[:, hdim:2 * hdim] + gh[:, hdim:2 * hdim])
        n = jnp.tanh(gi[:, 2 * hdim:] + r * (gh[:, 2 * hdim:] + bn_ref[...]))
        return (1.0 - z) * n + z * h

    def substep(hf, hb, gi_f, gi_b):
        gh_f = jnp.dot(hf.astype(jnp.bfloat16), whtf_ref[...],
                       preferred_element_type=jnp.float32)
        gh_b = jnp.dot(hb.astype(jnp.bfloat16), whtb_ref[...],
                       preferred_element_type=jnp.float32)
        return (gates(gi_f, gh_f, bnf_ref, hf),
                gates(gi_b, gh_b, bnb_ref, hb))

    hf, hb = substep(hf_ref[...], hb_ref[...], gi_f_ref[0], gi_b_ref[1])
    hf, hb = substep(hf, hb, gi_f_ref[1], gi_b_ref[0])
    hf_ref[...] = hf
    hb_ref[...] = hb

    @pl.when(t == nsteps - 1)
    def _head():
        hdim = hf.shape[1]
        wf = fcw_ref[0, :hdim][None, :]
        wb = fcw_ref[0, hdim:][None, :]
        logit = (jnp.sum(hf * wf, axis=1, keepdims=True)
                 + jnp.sum(hb * wb, axis=1, keepdims=True)
                 + fcb_ref[0, 0])
        out_ref[...] = jax.nn.sigmoid(logit)


def _gru_scan(gi, whtf, whtb, bnf, bnb, fcw, fcb, t_len, b_dim, h_dim):
    g3 = 3 * h_dim
    hsteps = t_len // 2
    return pl.pallas_call(
        _gru_scan_body,
        grid=(hsteps,),
        in_specs=[
            pl.BlockSpec((2, b_dim, g3), lambda t: (t, 0, 0)),
            pl.BlockSpec((2, b_dim, g3), lambda t: (hsteps - 1 - t, 0, 1)),
            pl.BlockSpec((h_dim, g3), lambda t: (0, 0)),
            pl.BlockSpec((h_dim, g3), lambda t: (0, 0)),
            pl.BlockSpec((1, h_dim), lambda t: (0, 0)),
            pl.BlockSpec((1, h_dim), lambda t: (0, 0)),
            pl.BlockSpec((1, 2 * h_dim), lambda t: (0, 0)),
            pl.BlockSpec((1, 1), lambda t: (0, 0)),
        ],
        out_specs=pl.BlockSpec((b_dim, 1), lambda t: (0, 0)),
        out_shape=jax.ShapeDtypeStruct((b_dim, 1), jnp.float32),
        scratch_shapes=[
            pltpu.VMEM((b_dim, h_dim), jnp.float32),
            pltpu.VMEM((b_dim, h_dim), jnp.float32),
        ],
        compiler_params=pltpu.CompilerParams(
            dimension_semantics=("arbitrary",),
        ),
    )(gi, gi, whtf, whtb, bnf, bnb, fcw, fcb)


def kernel(input, emb, Wih_f, Whh_f, bih_f, bhh_f, Wih_b, Whh_b, bih_b, bhh_b,
           fcW, fcb):
    b_dim, t_len = input.shape
    v_dim, e_dim = emb.shape
    h_dim = Whh_f.shape[1]
    n_idx = b_dim * t_len

    e_pad = (e_dim + 127) // 128 * 128
    emb_p = _pad_table(emb, e_pad)
    idx3d = input.T.reshape(_NW, n_idx // (_NW * _CHUNK), _CHUNK).astype(jnp.int32)
    x = _sc_gather(emb_p, idx3d, n_idx, e_pad)

    def fold_bias(bih, bhh):
        return jnp.concatenate(
            [bih[: 2 * h_dim] + bhh[: 2 * h_dim], bih[2 * h_dim:]])

    wcat = jnp.concatenate([Wih_f.T, Wih_b.T], axis=1)
    wcat = jnp.pad(wcat, ((0, e_pad - e_dim), (0, 0)))
    wcat = wcat.astype(jnp.bfloat16)
    bcat = jnp.concatenate([fold_bias(bih_f, bhh_f),
                            fold_bias(bih_b, bhh_b)])[None, :]
    gi = _input_matmul(x, wcat, bcat)
    gi = gi.reshape(t_len, b_dim, 6 * h_dim)

    label = _gru_scan(
        gi,
        Whh_f.T.astype(jnp.bfloat16), Whh_b.T.astype(jnp.bfloat16),
        bhh_f[2 * h_dim:][None, :], bhh_b[2 * h_dim:][None, :],
        fcW, fcb.reshape(1, 1),
        t_len, b_dim, h_dim,
    )
    return jnp.squeeze(label, axis=1)


# speedup vs baseline: 2.3957x; 1.0706x over previous
import functools

import jax
import jax.numpy as jnp
from jax import lax
from jax.experimental import pallas as pl
from jax.experimental.pallas import tpu as pltpu
from jax.experimental.pallas import tpu_sc as plsc


_NC = 2
_NS = 16
_NW = _NC * _NS
_CHUNK = 80


def _pad_table_body(x_ref, o_ref):
    bm = x_ref.shape[0]
    o_ref[...] = jnp.concatenate(
        [x_ref[...], jnp.zeros((bm, 84), jnp.float32)], axis=1)


def _pad_table(emb, e_pad, bm=2000):
    v_rows, e_dim = emb.shape
    return pl.pallas_call(
        _pad_table_body,
        grid=(v_rows // bm,),
        in_specs=[pl.BlockSpec((bm, e_dim), lambda i: (i, 0))],
        out_specs=pl.BlockSpec((bm, e_pad), lambda i: (i, 0)),
        out_shape=jax.ShapeDtypeStruct((v_rows, e_pad), jnp.float32),
        compiler_params=pltpu.CompilerParams(
            dimension_semantics=("arbitrary",),
        ),
    )(emb)


def _sc_gather(emb_p, idx3d, n_idx, e_pad):
    rows_per_w = n_idx // _NW
    chunks_per_w = rows_per_w // _CHUNK
    mesh = plsc.VectorSubcoreMesh(core_axis_name="c", subcore_axis_name="s")

    @functools.partial(
        pl.kernel,
        mesh=mesh,
        out_type=jax.ShapeDtypeStruct((n_idx, e_pad), jnp.float32),
        scratch_types=[
            pltpu.VMEM((chunks_per_w, _CHUNK), jnp.int32),
            pltpu.VMEM((_CHUNK, e_pad), jnp.float32),
            pltpu.SemaphoreType.DMA,
        ],
    )
    def gather_kernel(emb_hbm, idx_hbm, out_hbm, idx_v, rows_v, sem):
        wid = lax.axis_index("s") * _NC + lax.axis_index("c")
        base = wid * rows_per_w
        pltpu.sync_copy(idx_hbm.at[wid], idx_v)
        for c in range(chunks_per_w):
            pltpu.async_copy(emb_hbm.at[idx_v.at[c]], rows_v, sem).wait()
            pltpu.sync_copy(rows_v, out_hbm.at[pl.ds(base + c * _CHUNK, _CHUNK)])

    return gather_kernel(emb_p, idx3d)


def _input_matmul_body(x_ref, w_ref, b_ref, o_ref):
    o_ref[...] = (
        jnp.dot(x_ref[...].astype(jnp.bfloat16), w_ref[...],
                preferred_element_type=jnp.float32)
        + b_ref[...]
    ).astype(jnp.bfloat16)


def _input_matmul(x, wcat, bcat, bm=512):
    m, k = x.shape
    kw, n = wcat.shape
    return pl.pallas_call(
        _input_matmul_body,
        grid=(m // bm,),
        in_specs=[
            pl.BlockSpec((bm, k), lambda i: (i, 0)),
            pl.BlockSpec((kw, n), lambda i: (0, 0)),
            pl.BlockSpec((1, n), lambda i: (0, 0)),
        ],
        out_specs=pl.BlockSpec((bm, n), lambda i: (i, 0)),
        out_shape=jax.ShapeDtypeStruct((m, n), jnp.bfloat16),
        compiler_params=pltpu.CompilerParams(
            dimension_semantics=("arbitrary",),
        ),
    )(x, wcat, bcat)


def _gru_scan_body(gi_f_ref, gi_b_ref, whtf_ref, whtb_ref, bnf_ref, bnb_ref,
                   fcw_ref, fcb_ref, out_ref, hf_ref, hb_ref):
    t = pl.program_id(0)
    nsteps = pl.num_programs(0)

    @pl.when(t == 0)
    def _init():
        hf_ref[...] = jnp.zeros_like(hf_ref)
        hb_ref[...] = jnp.zeros_like(hb_ref)

    def gates(gi, gh, bn_ref, h):
        hdim = h.shape[1]
        gi = gi.astype(jnp.float32)
        r = jax.nn.sigmoid(gi[:, :hdim] + gh[:, :hdim])
        z = jax.nn.sigmoid(gi[:, hdim:2 * hdim] + gh[:, hdim:2 * hdim])
        n = jnp.tanh(gi[:, 2 * hdim:] + r * (gh[:, 2 * hdim:] + bn_ref[...]))
        return (1.0 - z) * n + z * h

    def substep(hf, hb, gi_f, gi_b):
        gh_f = jnp.dot(hf.astype(jnp.bfloat16), whtf_ref[...],
                       preferred_element_type=jnp.float32)
        gh_b = jnp.dot(hb.astype(jnp.bfloat16), whtb_ref[...],
                       preferred_element_type=jnp.float32)
        return (gates(gi_f, gh_f, bnf_ref, hf),
                gates(gi_b, gh_b, bnb_ref, hb))

    hf, hb = substep(hf_ref[...], hb_ref[...], gi_f_ref[0], gi_b_ref[1])
    hf, hb = substep(hf, hb, gi_f_ref[1], gi_b_ref[0])
    hf_ref[...] = hf
    hb_ref[...] = hb

    @pl.when(t == nsteps - 1)
    def _head():
        hdim = hf.shape[1]
        wf = fcw_ref[0, :hdim][None, :]
        wb = fcw_ref[0, hdim:][None, :]
        logit = (jnp.sum(hf * wf, axis=1, keepdims=True)
                 + jnp.sum(hb * wb, axis=1, keepdims=True)
                 + fcb_ref[0, 0])
        out_ref[...] = jax.nn.sigmoid(logit)


def _gru_scan(gi, whtf, whtb, bnf, bnb, fcw, fcb, t_len, b_dim, h_dim):
    g3 = 3 * h_dim
    hsteps = t_len // 2
    return pl.pallas_call(
        _gru_scan_body,
        grid=(hsteps,),
        in_specs=[
            pl.BlockSpec((2, b_dim, g3), lambda t: (t, 0, 0)),
            pl.BlockSpec((2, b_dim, g3), lambda t: (hsteps - 1 - t, 0, 1)),
            pl.BlockSpec((h_dim, g3), lambda t: (0, 0)),
            pl.BlockSpec((h_dim, g3), lambda t: (0, 0)),
            pl.BlockSpec((1, h_dim), lambda t: (0, 0)),
            pl.BlockSpec((1, h_dim), lambda t: (0, 0)),
            pl.BlockSpec((1, 2 * h_dim), lambda t: (0, 0)),
            pl.BlockSpec((1, 1), lambda t: (0, 0)),
        ],
        out_specs=pl.BlockSpec((b_dim, 1), lambda t: (0, 0)),
        out_shape=jax.ShapeDtypeStruct((b_dim, 1), jnp.float32),
        scratch_shapes=[
            pltpu.VMEM((b_dim, h_dim), jnp.float32),
            pltpu.VMEM((b_dim, h_dim), jnp.float32),
        ],
        compiler_params=pltpu.CompilerParams(
            dimension_semantics=("arbitrary",),
        ),
    )(gi, gi, whtf, whtb, bnf, bnb, fcw, fcb)


def kernel(input, emb, Wih_f, Whh_f, bih_f, bhh_f, Wih_b, Whh_b, bih_b, bhh_b,
           fcW, fcb):
    b_dim, t_len = input.shape
    v_dim, e_dim = emb.shape
    h_dim = Whh_f.shape[1]
    n_idx = b_dim * t_len

    e_pad = (e_dim + 127) // 128 * 128
    emb_p = _pad_table(emb, e_pad)
    idx3d = input.T.reshape(_NW, n_idx // (_NW * _CHUNK), _CHUNK).astype(jnp.int32)
    x = _sc_gather(emb_p, idx3d, n_idx, e_pad)

    def fold_bias(bih, bhh):
        return jnp.concatenate(
            [bih[: 2 * h_dim] + bhh[: 2 * h_dim], bih[2 * h_dim:]])

    wcat = jnp.concatenate([Wih_f.T, Wih_b.T], axis=1)
    wcat = jnp.pad(wcat, ((0, e_pad - e_dim), (0, 0)))
    wcat = wcat.astype(jnp.bfloat16)
    bcat = jnp.concatenate([fold_bias(bih_f, bhh_f),
                            fold_bias(bih_b, bhh_b)])[None, :]
    gi = _input_matmul(x, wcat, bcat)
    gi = gi.reshape(t_len, b_dim, 6 * h_dim)

    label = _gru_scan(
        gi,
        Whh_f.T.astype(jnp.bfloat16), Whh_b.T.astype(jnp.bfloat16),
        bhh_f[2 * h_dim:][None, :], bhh_b[2 * h_dim:][None, :],
        fcW, fcb.reshape(1, 1),
        t_len, b_dim, h_dim,
    )
    return jnp.squeeze(label, axis=1)


# speedup vs baseline: 2.4644x; 1.0287x over previous
import functools

import jax
import jax.numpy as jnp
from jax import lax
from jax.experimental import pallas as pl
from jax.experimental.pallas import tpu as pltpu
from jax.experimental.pallas import tpu_sc as plsc


_NC = 2
_NS = 16
_NW = _NC * _NS
_CHUNK = 80


def _pad_table_body(x_ref, o_ref):
    bm = x_ref.shape[0]
    o_ref[...] = jnp.concatenate(
        [x_ref[...], jnp.zeros((bm, 84), jnp.float32)], axis=1)


def _pad_table(emb, e_pad, bm=4000):
    v_rows, e_dim = emb.shape
    return pl.pallas_call(
        _pad_table_body,
        grid=(v_rows // bm,),
        in_specs=[pl.BlockSpec((bm, e_dim), lambda i: (i, 0))],
        out_specs=pl.BlockSpec((bm, e_pad), lambda i: (i, 0)),
        out_shape=jax.ShapeDtypeStruct((v_rows, e_pad), jnp.float32),
        compiler_params=pltpu.CompilerParams(
            dimension_semantics=("arbitrary",),
        ),
    )(emb)


def _sc_gather(emb_p, idx3d, n_idx, e_pad):
    rows_per_w = n_idx // _NW
    chunks_per_w = rows_per_w // _CHUNK
    mesh = plsc.VectorSubcoreMesh(core_axis_name="c", subcore_axis_name="s")

    @functools.partial(
        pl.kernel,
        mesh=mesh,
        out_type=jax.ShapeDtypeStruct((n_idx, e_pad), jnp.float32),
        scratch_types=[
            pltpu.VMEM((chunks_per_w, _CHUNK), jnp.int32),
            pltpu.VMEM((_CHUNK, e_pad), jnp.float32),
            pltpu.SemaphoreType.DMA,
        ],
    )
    def gather_kernel(emb_hbm, idx_hbm, out_hbm, idx_v, rows_v, sem):
        wid = lax.axis_index("s") * _NC + lax.axis_index("c")
        base = wid * rows_per_w
        pltpu.sync_copy(idx_hbm.at[wid], idx_v)
        for c in range(chunks_per_w):
            pltpu.async_copy(emb_hbm.at[idx_v.at[c]], rows_v, sem).wait()
            pltpu.sync_copy(rows_v, out_hbm.at[pl.ds(base + c * _CHUNK, _CHUNK)])

    return gather_kernel(emb_p, idx3d)


def _input_matmul_body(x_ref, w_ref, b_ref, o_ref):
    o_ref[...] = (
        jnp.dot(x_ref[...].astype(jnp.bfloat16), w_ref[...],
                preferred_element_type=jnp.float32)
        + b_ref[...]
    ).astype(jnp.bfloat16)


def _input_matmul(x, wcat, bcat, bm=512):
    m, k = x.shape
    kw, n = wcat.shape
    return pl.pallas_call(
        _input_matmul_body,
        grid=(m // bm,),
        in_specs=[
            pl.BlockSpec((bm, k), lambda i: (i, 0)),
            pl.BlockSpec((kw, n), lambda i: (0, 0)),
            pl.BlockSpec((1, n), lambda i: (0, 0)),
        ],
        out_specs=pl.BlockSpec((bm, n), lambda i: (i, 0)),
        out_shape=jax.ShapeDtypeStruct((m, n), jnp.bfloat16),
        compiler_params=pltpu.CompilerParams(
            dimension_semantics=("arbitrary",),
        ),
    )(x, wcat, bcat)


def _gru_scan_body(gi_f_ref, gi_b_ref, whtf_ref, whtb_ref, bnf_ref, bnb_ref,
                   fcw_ref, fcb_ref, out_ref, hf_ref, hb_ref):
    t = pl.program_id(0)
    nsteps = pl.num_programs(0)

    @pl.when(t == 0)
    def _init():
        hf_ref[...] = jnp.zeros_like(hf_ref)
        hb_ref[...] = jnp.zeros_like(hb_ref)

    def gates(gi, gh, bn_ref, h):
        hdim = h.shape[1]
        gi = gi.astype(jnp.float32)
        r = jax.nn.sigmoid(gi[:, :hdim] + gh[:, :hdim])
        z = jax.nn.sigmoid(gi[:, hdim:2 * hdim] + gh[:, hdim:2 * hdim])
        n = jnp.tanh(gi[:, 2 * hdim:] + r * (gh[:, 2 * hdim:] + bn_ref[...]))
        return (1.0 - z) * n + z * h

    def substep(hf, hb, gi_f, gi_b):
        gh_f = jnp.dot(hf.astype(jnp.bfloat16), whtf_ref[...],
                       preferred_element_type=jnp.float32)
        gh_b = jnp.dot(hb.astype(jnp.bfloat16), whtb_ref[...],
                       preferred_element_type=jnp.float32)
        return (gates(gi_f, gh_f, bnf_ref, hf),
                gates(gi_b, gh_b, bnb_ref, hb))

    unroll = gi_f_ref.shape[0]
    hf, hb = hf_ref[...], hb_ref[...]
    for u in range(unroll):
        hf, hb = substep(hf, hb, gi_f_ref[u], gi_b_ref[unroll - 1 - u])
    hf_ref[...] = hf
    hb_ref[...] = hb

    @pl.when(t == nsteps - 1)
    def _head():
        hdim = hf.shape[1]
        wf = fcw_ref[0, :hdim][None, :]
        wb = fcw_ref[0, hdim:][None, :]
        logit = (jnp.sum(hf * wf, axis=1, keepdims=True)
                 + jnp.sum(hb * wb, axis=1, keepdims=True)
                 + fcb_ref[0, 0])
        out_ref[...] = jax.nn.sigmoid(logit)


def _gru_scan(gi, whtf, whtb, bnf, bnb, fcw, fcb, t_len, b_dim, h_dim):
    g3 = 3 * h_dim
    unroll = 4
    hsteps = t_len // unroll
    return pl.pallas_call(
        _gru_scan_body,
        grid=(hsteps,),
        in_specs=[
            pl.BlockSpec((unroll, b_dim, g3), lambda t: (t, 0, 0)),
            pl.BlockSpec((unroll, b_dim, g3), lambda t: (hsteps - 1 - t, 0, 1)),
            pl.BlockSpec((h_dim, g3), lambda t: (0, 0)),
            pl.BlockSpec((h_dim, g3), lambda t: (0, 0)),
            pl.BlockSpec((1, h_dim), lambda t: (0, 0)),
            pl.BlockSpec((1, h_dim), lambda t: (0, 0)),
            pl.BlockSpec((1, 2 * h_dim), lambda t: (0, 0)),
            pl.BlockSpec((1, 1), lambda t: (0, 0)),
        ],
        out_specs=pl.BlockSpec((b_dim, 1), lambda t: (0, 0)),
        out_shape=jax.ShapeDtypeStruct((b_dim, 1), jnp.float32),
        scratch_shapes=[
            pltpu.VMEM((b_dim, h_dim), jnp.float32),
            pltpu.VMEM((b_dim, h_dim), jnp.float32),
        ],
        compiler_params=pltpu.CompilerParams(
            dimension_semantics=("arbitrary",),
        ),
    )(gi, gi, whtf, whtb, bnf, bnb, fcw, fcb)


def kernel(input, emb, Wih_f, Whh_f, bih_f, bhh_f, Wih_b, Whh_b, bih_b, bhh_b,
           fcW, fcb):
    b_dim, t_len = input.shape
    v_dim, e_dim = emb.shape
    h_dim = Whh_f.shape[1]
    n_idx = b_dim * t_len

    e_pad = (e_dim + 127) // 128 * 128
    emb_p = _pad_table(emb, e_pad)
    idx3d = input.T.reshape(_NW, n_idx // (_NW * _CHUNK), _CHUNK).astype(jnp.int32)
    x = _sc_gather(emb_p, idx3d, n_idx, e_pad)

    def fold_bias(bih, bhh):
        return jnp.concatenate(
            [bih[: 2 * h_dim] + bhh[: 2 * h_dim], bih[2 * h_dim:]])

    wcat = jnp.concatenate([Wih_f.T, Wih_b.T], axis=1)
    wcat = jnp.pad(wcat, ((0, e_pad - e_dim), (0, 0)))
    wcat = wcat.astype(jnp.bfloat16)
    bcat = jnp.concatenate([fold_bias(bih_f, bhh_f),
                            fold_bias(bih_b, bhh_b)])[None, :]
    gi = _input_matmul(x, wcat, bcat)
    gi = gi.reshape(t_len, b_dim, 6 * h_dim)

    label = _gru_scan(
        gi,
        Whh_f.T.astype(jnp.bfloat16), Whh_b.T.astype(jnp.bfloat16),
        bhh_f[2 * h_dim:][None, :], bhh_b[2 * h_dim:][None, :],
        fcW, fcb.reshape(1, 1),
        t_len, b_dim, h_dim,
    )
    return jnp.squeeze(label, axis=1)


# speedup vs baseline: 2.4751x; 1.0043x over previous
import functools

import jax
import jax.numpy as jnp
from jax import lax
from jax.experimental import pallas as pl
from jax.experimental.pallas import tpu as pltpu
from jax.experimental.pallas import tpu_sc as plsc


_NC = 2
_NS = 16
_NW = _NC * _NS
_CHUNK = 80


def _pad_table_body(x_ref, o_ref):
    bm = x_ref.shape[0]
    o_ref[...] = jnp.concatenate(
        [x_ref[...], jnp.zeros((bm, 84), jnp.float32)], axis=1)


def _pad_table(emb, e_pad, bm=4000):
    v_rows, e_dim = emb.shape
    return pl.pallas_call(
        _pad_table_body,
        grid=(v_rows // bm,),
        in_specs=[pl.BlockSpec((bm, e_dim), lambda i: (i, 0))],
        out_specs=pl.BlockSpec((bm, e_pad), lambda i: (i, 0)),
        out_shape=jax.ShapeDtypeStruct((v_rows, e_pad), jnp.float32),
        compiler_params=pltpu.CompilerParams(
            dimension_semantics=("arbitrary",),
        ),
    )(emb)


def _sc_gather(emb_p, idx3d, n_idx, e_pad):
    rows_per_w = n_idx // _NW
    chunks_per_w = rows_per_w // _CHUNK
    mesh = plsc.VectorSubcoreMesh(core_axis_name="c", subcore_axis_name="s")

    @functools.partial(
        pl.kernel,
        mesh=mesh,
        out_type=jax.ShapeDtypeStruct((n_idx, e_pad), jnp.float32),
        scratch_types=[
            pltpu.VMEM((chunks_per_w, _CHUNK), jnp.int32),
            pltpu.VMEM((_CHUNK, e_pad), jnp.float32),
            pltpu.SemaphoreType.DMA,
        ],
    )
    def gather_kernel(emb_hbm, idx_hbm, out_hbm, idx_v, rows_v, sem):
        wid = lax.axis_index("s") * _NC + lax.axis_index("c")
        base = wid * rows_per_w
        pltpu.sync_copy(idx_hbm.at[wid], idx_v)
        for c in range(chunks_per_w):
            pltpu.async_copy(emb_hbm.at[idx_v.at[c]], rows_v, sem).wait()
            pltpu.sync_copy(rows_v, out_hbm.at[pl.ds(base + c * _CHUNK, _CHUNK)])

    return gather_kernel(emb_p, idx3d)


def _input_matmul_body(x_ref, w_ref, b_ref, o_ref):
    o_ref[...] = (
        jnp.dot(x_ref[...].astype(jnp.bfloat16), w_ref[...],
                preferred_element_type=jnp.float32)
        + b_ref[...]
    ).astype(jnp.bfloat16)


def _input_matmul(x, wcat, bcat, bm=512):
    m, k = x.shape
    kw, n = wcat.shape
    return pl.pallas_call(
        _input_matmul_body,
        grid=(m // bm,),
        in_specs=[
            pl.BlockSpec((bm, k), lambda i: (i, 0)),
            pl.BlockSpec((kw, n), lambda i: (0, 0)),
            pl.BlockSpec((1, n), lambda i: (0, 0)),
        ],
        out_specs=pl.BlockSpec((bm, n), lambda i: (i, 0)),
        out_shape=jax.ShapeDtypeStruct((m, n), jnp.bfloat16),
        compiler_params=pltpu.CompilerParams(
            dimension_semantics=("arbitrary",),
        ),
    )(x, wcat, bcat)


def _gru_scan_body(gi_f_ref, gi_b_ref, whtf_ref, whtb_ref, bnf_ref, bnb_ref,
                   fcw_ref, fcb_ref, out_ref, hf_ref, hb_ref):
    t = pl.program_id(0)
    nsteps = pl.num_programs(0)

    @pl.when(t == 0)
    def _init():
        hf_ref[...] = jnp.zeros_like(hf_ref)
        hb_ref[...] = jnp.zeros_like(hb_ref)

    def gates(gi, gh, bn_ref, h):
        hdim = h.shape[1]
        gi = gi.astype(jnp.float32)
        r = jax.nn.sigmoid(gi[:, :hdim] + gh[:, :hdim])
        z = jax.nn.sigmoid(gi[:, hdim:2 * hdim] + gh[:, hdim:2 * hdim])
        n = jnp.tanh(gi[:, 2 * hdim:] + r * (gh[:, 2 * hdim:] + bn_ref[...]))
        return (1.0 - z) * n + z * h

    def substep(hf, hb, gi_f, gi_b):
        gh_f = jnp.dot(hf.astype(jnp.bfloat16), whtf_ref[...],
                       preferred_element_type=jnp.float32)
        gh_b = jnp.dot(hb.astype(jnp.bfloat16), whtb_ref[...],
                       preferred_element_type=jnp.float32)
        return (gates(gi_f, gh_f, bnf_ref, hf),
                gates(gi_b, gh_b, bnb_ref, hb))

    unroll = gi_f_ref.shape[0]
    hf, hb = hf_ref[...], hb_ref[...]
    for u in range(unroll):
        hf, hb = substep(hf, hb, gi_f_ref[u], gi_b_ref[unroll - 1 - u])
    hf_ref[...] = hf
    hb_ref[...] = hb

    @pl.when(t == nsteps - 1)
    def _head():
        hdim = hf.shape[1]
        wf = fcw_ref[0, :hdim][None, :]
        wb = fcw_ref[0, hdim:][None, :]
        logit = (jnp.sum(hf * wf, axis=1, keepdims=True)
                 + jnp.sum(hb * wb, axis=1, keepdims=True)
                 + fcb_ref[0, 0])
        out_ref[...] = jax.nn.sigmoid(logit)


def _gru_scan(gi, whtf, whtb, bnf, bnb, fcw, fcb, t_len, b_dim, h_dim):
    g3 = 3 * h_dim
    unroll = 8
    hsteps = t_len // unroll
    return pl.pallas_call(
        _gru_scan_body,
        grid=(hsteps,),
        in_specs=[
            pl.BlockSpec((unroll, b_dim, g3), lambda t: (t, 0, 0)),
            pl.BlockSpec((unroll, b_dim, g3), lambda t: (hsteps - 1 - t, 0, 1)),
            pl.BlockSpec((h_dim, g3), lambda t: (0, 0)),
            pl.BlockSpec((h_dim, g3), lambda t: (0, 0)),
            pl.BlockSpec((1, h_dim), lambda t: (0, 0)),
            pl.BlockSpec((1, h_dim), lambda t: (0, 0)),
            pl.BlockSpec((1, 2 * h_dim), lambda t: (0, 0)),
            pl.BlockSpec((1, 1), lambda t: (0, 0)),
        ],
        out_specs=pl.BlockSpec((b_dim, 1), lambda t: (0, 0)),
        out_shape=jax.ShapeDtypeStruct((b_dim, 1), jnp.float32),
        scratch_shapes=[
            pltpu.VMEM((b_dim, h_dim), jnp.float32),
            pltpu.VMEM((b_dim, h_dim), jnp.float32),
        ],
        compiler_params=pltpu.CompilerParams(
            dimension_semantics=("arbitrary",),
        ),
    )(gi, gi, whtf, whtb, bnf, bnb, fcw, fcb)


def kernel(input, emb, Wih_f, Whh_f, bih_f, bhh_f, Wih_b, Whh_b, bih_b, bhh_b,
           fcW, fcb):
    b_dim, t_len = input.shape
    v_dim, e_dim = emb.shape
    h_dim = Whh_f.shape[1]
    n_idx = b_dim * t_len

    e_pad = (e_dim + 127) // 128 * 128
    emb_p = _pad_table(emb, e_pad)
    idx3d = input.T.reshape(_NW, n_idx // (_NW * _CHUNK), _CHUNK).astype(jnp.int32)
    x = _sc_gather(emb_p, idx3d, n_idx, e_pad)

    def fold_bias(bih, bhh):
        return jnp.concatenate(
            [bih[: 2 * h_dim] + bhh[: 2 * h_dim], bih[2 * h_dim:]])

    wcat = jnp.concatenate([Wih_f.T, Wih_b.T], axis=1)
    wcat = jnp.pad(wcat, ((0, e_pad - e_dim), (0, 0)))
    wcat = wcat.astype(jnp.bfloat16)
    bcat = jnp.concatenate([fold_bias(bih_f, bhh_f),
                            fold_bias(bih_b, bhh_b)])[None, :]
    gi = _input_matmul(x, wcat, bcat)
    gi = gi.reshape(t_len, b_dim, 6 * h_dim)

    label = _gru_scan(
        gi,
        Whh_f.T.astype(jnp.bfloat16), Whh_b.T.astype(jnp.bfloat16),
        bhh_f[2 * h_dim:][None, :], bhh_b[2 * h_dim:][None, :],
        fcW, fcb.reshape(1, 1),
        t_len, b_dim, h_dim,
    )
    return jnp.squeeze(label, axis=1)


# speedup vs baseline: 2.4782x; 1.0013x over previous
import functools

import jax
import jax.numpy as jnp
from jax import lax
from jax.experimental import pallas as pl
from jax.experimental.pallas import tpu as pltpu
from jax.experimental.pallas import tpu_sc as plsc


_NC = 2
_NS = 16
_NW = _NC * _NS
_CHUNK = 80


def _pad_table_body(x_ref, o_ref):
    bm, e_dim = x_ref.shape
    pad = o_ref.shape[1] - e_dim
    o_ref[...] = jnp.concatenate(
        [x_ref[...], jnp.zeros((bm, pad), jnp.float32)], axis=1)


def _pad_table(emb, e_pad, bm=4000):
    v_rows, e_dim = emb.shape
    return pl.pallas_call(
        _pad_table_body,
        grid=(v_rows // bm,),
        in_specs=[pl.BlockSpec((bm, e_dim), lambda i: (i, 0))],
        out_specs=pl.BlockSpec((bm, e_pad), lambda i: (i, 0)),
        out_shape=jax.ShapeDtypeStruct((v_rows, e_pad), jnp.float32),
        compiler_params=pltpu.CompilerParams(
            dimension_semantics=("arbitrary",),
        ),
    )(emb)


def _sc_gather(emb_p, idx3d, n_idx, e_pad):
    rows_per_w = n_idx // _NW
    chunks_per_w = rows_per_w // _CHUNK
    mesh = plsc.VectorSubcoreMesh(core_axis_name="c", subcore_axis_name="s")

    @functools.partial(
        pl.kernel,
        mesh=mesh,
        out_type=jax.ShapeDtypeStruct((n_idx, e_pad), jnp.float32),
        scratch_types=[
            pltpu.VMEM((chunks_per_w, _CHUNK), jnp.int32),
            pltpu.VMEM((_CHUNK, e_pad), jnp.float32),
            pltpu.SemaphoreType.DMA,
        ],
    )
    def gather_kernel(emb_hbm, idx_hbm, out_hbm, idx_v, rows_v, sem):
        wid = lax.axis_index("s") * _NC + lax.axis_index("c")
        base = wid * rows_per_w
        pltpu.sync_copy(idx_hbm.at[wid], idx_v)
        for c in range(chunks_per_w):
            pltpu.async_copy(emb_hbm.at[idx_v.at[c]], rows_v, sem).wait()
            pltpu.sync_copy(rows_v, out_hbm.at[pl.ds(base + c * _CHUNK, _CHUNK)])

    return gather_kernel(emb_p, idx3d)


def _input_matmul_body(x_ref, w_ref, b_ref, o_ref):
    o_ref[...] = (
        jnp.dot(x_ref[...].astype(jnp.bfloat16), w_ref[...],
                preferred_element_type=jnp.float32)
        + b_ref[...]
    ).astype(jnp.bfloat16)


def _input_matmul(x, wcat, bcat, bm=512):
    m, k = x.shape
    kw, n = wcat.shape
    return pl.pallas_call(
        _input_matmul_body,
        grid=(m // bm,),
        in_specs=[
            pl.BlockSpec((bm, k), lambda i: (i, 0)),
            pl.BlockSpec((kw, n), lambda i: (0, 0)),
            pl.BlockSpec((1, n), lambda i: (0, 0)),
        ],
        out_specs=pl.BlockSpec((bm, n), lambda i: (i, 0)),
        out_shape=jax.ShapeDtypeStruct((m, n), jnp.bfloat16),
        compiler_params=pltpu.CompilerParams(
            dimension_semantics=("arbitrary",),
        ),
    )(x, wcat, bcat)


def _gru_scan_body(gi_f_ref, gi_b_ref, whtf_ref, whtb_ref, bnf_ref, bnb_ref,
                   fcw_ref, fcb_ref, out_ref, hf_ref, hb_ref):
    t = pl.program_id(0)
    nsteps = pl.num_programs(0)

    @pl.when(t == 0)
    def _init():
        hf_ref[...] = jnp.zeros_like(hf_ref)
        hb_ref[...] = jnp.zeros_like(hb_ref)

    def gates(gi, gh, bn_ref, h):
        hdim = h.shape[1]
        gi = gi.astype(jnp.float32)
        r = jax.nn.sigmoid(gi[:, :hdim] + gh[:, :hdim])
        z = jax.nn.sigmoid(gi[:, hdim:2 * hdim] + gh[:, hdim:2 * hdim])
        n = jnp.tanh(gi[:, 2 * hdim:] + r * (gh[:, 2 * hdim:] + bn_ref[...]))
        return (1.0 - z) * n + z * h

    def substep(hf, hb, gi_f, gi_b):
        gh_f = jnp.dot(hf.astype(jnp.bfloat16), whtf_ref[...],
                       preferred_element_type=jnp.float32)
        gh_b = jnp.dot(hb.astype(jnp.bfloat16), whtb_ref[...],
                       preferred_element_type=jnp.float32)
        return (gates(gi_f, gh_f, bnf_ref, hf),
                gates(gi_b, gh_b, bnb_ref, hb))

    unroll = gi_f_ref.shape[0]
    hf, hb = hf_ref[...], hb_ref[...]
    for u in range(unroll):
        hf, hb = substep(hf, hb, gi_f_ref[u], gi_b_ref[unroll - 1 - u])
    hf_ref[...] = hf
    hb_ref[...] = hb

    @pl.when(t == nsteps - 1)
    def _head():
        hdim = hf.shape[1]
        wf = fcw_ref[0, :hdim][None, :]
        wb = fcw_ref[0, hdim:][None, :]
        logit = (jnp.sum(hf * wf, axis=1, keepdims=True)
                 + jnp.sum(hb * wb, axis=1, keepdims=True)
                 + fcb_ref[0, 0])
        out_ref[...] = jax.nn.sigmoid(logit)


def _gru_scan(gi, whtf, whtb, bnf, bnb, fcw, fcb, t_len, b_dim, h_dim):
    g3 = 3 * h_dim
    unroll = 8
    hsteps = t_len // unroll
    return pl.pallas_call(
        _gru_scan_body,
        grid=(hsteps,),
        in_specs=[
            pl.BlockSpec((unroll, b_dim, g3), lambda t: (t, 0, 0)),
            pl.BlockSpec((unroll, b_dim, g3), lambda t: (hsteps - 1 - t, 0, 1)),
            pl.BlockSpec((h_dim, g3), lambda t: (0, 0)),
            pl.BlockSpec((h_dim, g3), lambda t: (0, 0)),
            pl.BlockSpec((1, h_dim), lambda t: (0, 0)),
            pl.BlockSpec((1, h_dim), lambda t: (0, 0)),
            pl.BlockSpec((1, 2 * h_dim), lambda t: (0, 0)),
            pl.BlockSpec((1, 1), lambda t: (0, 0)),
        ],
        out_specs=pl.BlockSpec((b_dim, 1), lambda t: (0, 0)),
        out_shape=jax.ShapeDtypeStruct((b_dim, 1), jnp.float32),
        scratch_shapes=[
            pltpu.VMEM((b_dim, h_dim), jnp.float32),
            pltpu.VMEM((b_dim, h_dim), jnp.float32),
        ],
        compiler_params=pltpu.CompilerParams(
            dimension_semantics=("arbitrary",),
        ),
    )(gi, gi, whtf, whtb, bnf, bnb, fcw, fcb)


def kernel(input, emb, Wih_f, Whh_f, bih_f, bhh_f, Wih_b, Whh_b, bih_b, bhh_b,
           fcW, fcb):
    b_dim, t_len = input.shape
    v_dim, e_dim = emb.shape
    h_dim = Whh_f.shape[1]
    n_idx = b_dim * t_len

    e_pad = (e_dim + 127) // 128 * 128
    emb_p = _pad_table(emb, e_pad)
    idx3d = input.T.reshape(_NW, n_idx // (_NW * _CHUNK), _CHUNK).astype(jnp.int32)
    x = _sc_gather(emb_p, idx3d, n_idx, e_pad)

    def fold_bias(bih, bhh):
        return jnp.concatenate(
            [bih[: 2 * h_dim] + bhh[: 2 * h_dim], bih[2 * h_dim:]])

    wcat = jnp.concatenate([Wih_f.T, Wih_b.T], axis=1)
    wcat = jnp.pad(wcat, ((0, e_pad - e_dim), (0, 0)))
    wcat = wcat.astype(jnp.bfloat16)
    bcat = jnp.concatenate([fold_bias(bih_f, bhh_f),
                            fold_bias(bih_b, bhh_b)])[None, :]
    gi = _input_matmul(x, wcat, bcat)
    gi = gi.reshape(t_len, b_dim, 6 * h_dim)

    label = _gru_scan(
        gi,
        Whh_f.T.astype(jnp.bfloat16), Whh_b.T.astype(jnp.bfloat16),
        bhh_f[2 * h_dim:][None, :], bhh_b[2 * h_dim:][None, :],
        fcW, fcb.reshape(1, 1),
        t_len, b_dim, h_dim,
    )
    return jnp.squeeze(label, axis=1)
